# Initial kernel scaffold; baseline (speedup 1.0000x reference)
import jax
import jax.numpy as jnp
from jax.experimental import pallas as pl


def kernel(x, token):
    raise NotImplementedError("write your pallas kernel here")


# speedup vs baseline: 4.0812x; 4.0812x over previous
import functools

import jax
import jax.numpy as jnp
from jax import lax
from jax.experimental import pallas as pl
from jax.experimental.pallas import tpu as pltpu
from jax.experimental.pallas import tpu_sc as plsc

_INFO = plsc.get_sparse_core_info()
_NC = _INFO.num_cores
_NS = _INFO.num_subcores
_NW = _NC * _NS
_CHUNK = 128


def _make_gather(num_rows: int, d: int, b_total: int):
    assert b_total % (_NW * _CHUNK) == 0
    n_chunks = b_total // (_NW * _CHUNK)
    b_per_w = n_chunks * _CHUNK
    mesh = plsc.VectorSubcoreMesh(core_axis_name="c", subcore_axis_name="s")

    @functools.partial(
        pl.kernel,
        mesh=mesh,
        out_type=jax.ShapeDtypeStruct((b_total, d), jnp.float32),
        scratch_types=[
            pltpu.VMEM((n_chunks, _CHUNK), jnp.int32),
            pltpu.VMEM((_CHUNK, d), jnp.float32),
            pltpu.SemaphoreType.DMA,
        ],
        compiler_params=pltpu.CompilerParams(use_tc_tiling_on_sc=False),
    )
    def gather_kernel(token_hbm, idx_hbm, out_hbm, idx_v, rows_v, sem):
        wid = lax.axis_index("s") * _NC + lax.axis_index("c")
        base = wid * b_per_w
        pltpu.sync_copy(idx_hbm.at[wid], idx_v)

        def body(j, carry):
            pltpu.async_copy(token_hbm.at[idx_v.at[j]], rows_v, sem).wait()
            pltpu.sync_copy(rows_v, out_hbm.at[pl.ds(base + j * _CHUNK, _CHUNK)])
            return carry

        lax.fori_loop(0, n_chunks, body, 0)

    return gather_kernel


def kernel(x, token):
    b0, b1 = x.shape
    num_rows, d = token.shape
    b_total = b0 * b1
    idx = x.reshape(_NW, b_total // (_NW * _CHUNK), _CHUNK).astype(jnp.int32)
    out = _make_gather(num_rows, d, b_total)(token, idx)
    return out.reshape(b0, b1, d)


# speedup vs baseline: 4.6791x; 1.1465x over previous
import functools

import jax
import jax.numpy as jnp
from jax import lax
from jax.experimental import pallas as pl
from jax.experimental.pallas import tpu as pltpu
from jax.experimental.pallas import tpu_sc as plsc

_INFO = plsc.get_sparse_core_info()
_NC = _INFO.num_cores
_NS = _INFO.num_subcores
_NW = _NC * _NS
_CHUNK = 128
_NBUF = 5


def _make_gather(num_rows: int, d: int, b_total: int):
    assert b_total % (_NW * _CHUNK) == 0
    n_chunks = b_total // (_NW * _CHUNK)
    assert n_chunks % _NBUF == 0
    n_groups = n_chunks // _NBUF
    b_per_w = n_chunks * _CHUNK
    mesh = plsc.VectorSubcoreMesh(core_axis_name="c", subcore_axis_name="s")

    @functools.partial(
        pl.kernel,
        mesh=mesh,
        out_type=jax.ShapeDtypeStruct((b_total, d), jnp.float32),
        scratch_types=[
            pltpu.VMEM((n_chunks, _CHUNK), jnp.int32),
            pltpu.VMEM((_NBUF, _CHUNK, d), jnp.float32),
            pltpu.SemaphoreType.DMA((_NBUF,)),
        ],
        compiler_params=pltpu.CompilerParams(use_tc_tiling_on_sc=False),
    )
    def gather_kernel(token_hbm, idx_hbm, out_hbm, idx_v, rows_v, sems):
        wid = lax.axis_index("s") * _NC + lax.axis_index("c")
        base = wid * b_per_w
        pltpu.sync_copy(idx_hbm.at[wid], idx_v)

        def start_gather(j, b):
            pltpu.async_copy(token_hbm.at[idx_v.at[j]], rows_v.at[b], sems.at[b])

        def wait_gather(j, b):
            pltpu.make_async_copy(
                token_hbm.at[idx_v.at[j]], rows_v.at[b], sems.at[b]
            ).wait()

        for b in range(_NBUF):
            start_gather(b, b)

        def group(g, carry):
            for b in range(_NBUF):
                j = g * _NBUF + b
                wait_gather(j, b)
                pltpu.sync_copy(
                    rows_v.at[b], out_hbm.at[pl.ds(base + j * _CHUNK, _CHUNK)]
                )

                @pl.when(g < n_groups - 1)
                def _():
                    start_gather(j + _NBUF, b)

            return carry

        lax.fori_loop(0, n_groups, group, 0)

    return gather_kernel


def kernel(x, token):
    b0, b1 = x.shape
    num_rows, d = token.shape
    b_total = b0 * b1
    idx = x.reshape(_NW, b_total // (_NW * _CHUNK), _CHUNK).astype(jnp.int32)
    out = _make_gather(num_rows, d, b_total)(token, idx)
    return out.reshape(b0, b1, d)
